# baseline (speedup 1.0000x reference)
import functools

import jax
import jax.numpy as jnp
import numpy as np
from jax.experimental import pallas as pl
from jax.experimental.pallas import tpu as pltpu

D_MODEL = 1024
N_HEADS = 16
D_HEAD = 64
N_EXP = 8
ROPE_BASE = 10000.0
NEG_INF = -1e30


def _top2_gates(logits_planes):
    m1 = logits_planes[0]
    i1 = jnp.zeros_like(m1)
    for e in range(1, N_EXP):
        gt = logits_planes[e] > m1
        m1 = jnp.where(gt, logits_planes[e], m1)
        i1 = jnp.where(gt, float(e), i1)
    p0 = jnp.where(i1 == 0.0, NEG_INF, logits_planes[0])
    m2 = p0
    i2 = jnp.zeros_like(m1)
    for e in range(1, N_EXP):
        pe = jnp.where(i1 == float(e), NEG_INF, logits_planes[e])
        gt = pe > m2
        m2 = jnp.where(gt, pe, m2)
        i2 = jnp.where(gt, float(e), i2)
    g1 = jax.nn.sigmoid(m1)
    g2 = jax.nn.sigmoid(m2)
    gates = []
    for e in range(N_EXP):
        ge = jnp.where(i1 == float(e), g1, 0.0) + jnp.where(i2 == float(e), g2, 0.0)
        gates.append(ge)
    return gates


def _proj_kernel(x_ref, wqk_ref, wv_ref, wsel_ref, q_ref, k_ref, v_ref, go_ref,
                 *, tile):
    t = pl.program_id(0)
    x = x_ref[...]
    xb = x.astype(jnp.bfloat16)

    qk = jax.lax.dot_general(xb, wqk_ref[...], (((1,), (0,)), ((), ())),
                             preferred_element_type=jnp.float32)

    half = D_HEAD // 2
    rows = (jax.lax.broadcasted_iota(jnp.int32, (tile, half), 0) + t * tile).astype(jnp.float32)
    fidx = jax.lax.broadcasted_iota(jnp.int32, (tile, half), 1).astype(jnp.float32)
    inv_freq = jnp.exp(fidx * (-2.0 / D_HEAD * np.log(ROPE_BASE)))
    ang = rows * inv_freq
    c = jnp.cos(ang)
    s = jnp.sin(ang)

    for h in range(N_HEADS):
        qh = qk[:, h * D_HEAD:(h + 1) * D_HEAD]
        q1 = qh[:, :half]
        q2 = qh[:, half:]
        q_ref[h] = jnp.concatenate([q1 * c - q2 * s, q2 * c + q1 * s], axis=-1)
        kh = qk[:, (N_HEADS + h) * D_HEAD:(N_HEADS + h + 1) * D_HEAD]
        k1 = kh[:, :half]
        k2 = kh[:, half:]
        k_ref[h] = jnp.concatenate([k1 * c - k2 * s, k2 * c + k1 * s], axis=-1)

    lg = jax.lax.dot_general(x, wsel_ref[...], (((1,), (0,)), ((), ())),
                             preferred_element_type=jnp.float32)
    EH = N_EXP * N_HEADS
    planes_v = [lg[:, e * N_HEADS:(e + 1) * N_HEADS] for e in range(N_EXP)]
    planes_o = [lg[:, EH + e * N_HEADS:EH + (e + 1) * N_HEADS] for e in range(N_EXP)]
    gates_v = _top2_gates(planes_v)
    gates_o = _top2_gates(planes_o)
    for e in range(N_EXP):
        go_ref[:, e * N_HEADS:(e + 1) * N_HEADS] = gates_o[e]

    vall = jax.lax.dot_general(xb, wv_ref[...], (((1,), (0,)), ((), ())),
                               preferred_element_type=jnp.float32)
    for h in range(N_HEADS):
        acc = None
        for e in range(N_EXP):
            blk = vall[:, (h * N_EXP + e) * D_HEAD:(h * N_EXP + e + 1) * D_HEAD]
            g = gates_v[e][:, h:h + 1]
            term = blk * g
            acc = term if acc is None else acc + term
        v_ref[h] = acc


def _attn_kernel(q_ref, k_ref, v_ref, o_ref, *, tq, seq_len):
    qt = pl.program_id(1)
    q = q_ref[0]
    k = k_ref[0]
    v = v_ref[0]
    scale = 1.0 / np.sqrt(D_HEAD)
    scores = jax.lax.dot_general(q.astype(jnp.bfloat16), k.astype(jnp.bfloat16),
                                 (((1,), (1,)), ((), ())),
                                 preferred_element_type=jnp.float32)
    scores = scores * scale
    rows = jax.lax.broadcasted_iota(jnp.int32, (tq, seq_len), 0) + qt * tq
    cols = jax.lax.broadcasted_iota(jnp.int32, (tq, seq_len), 1)
    scores = jnp.where(cols <= rows, scores, NEG_INF)
    m = jnp.max(scores, axis=-1, keepdims=True)
    p = jnp.exp(scores - m)
    l = jnp.sum(p, axis=-1, keepdims=True)
    ctx = jax.lax.dot_general(p.astype(jnp.bfloat16), v.astype(jnp.bfloat16),
                              (((1,), (0,)), ((), ())),
                              preferred_element_type=jnp.float32)
    o_ref[0] = ctx / l


def _out_kernel(ctx_ref, go_ref, wo_ref, o_ref, scratch_ref):
    for h in range(N_HEADS):
        ch = ctx_ref[h]
        for e in range(N_EXP):
            g = go_ref[:, e * N_HEADS + h:e * N_HEADS + h + 1]
            scratch_ref[:, (h * N_EXP + e) * D_HEAD:(h * N_EXP + e + 1) * D_HEAD] = (
                (ch * g).astype(jnp.bfloat16))
    o_ref[...] = jax.lax.dot_general(scratch_ref[...], wo_ref[...],
                                     (((1,), (0,)), ((), ())),
                                     preferred_element_type=jnp.float32)


def kernel(token_stream, Wq, Wk, Wv, Wo, Wsel_v, Wsel_o):
    B, S, _ = token_stream.shape
    x = token_stream.reshape(S, D_MODEL)
    H, E, dh = N_HEADS, N_EXP, D_HEAD

    wqk = jnp.concatenate([Wq, Wk], axis=1).astype(jnp.bfloat16)
    wv_flat = Wv.transpose(2, 0, 1, 3).reshape(D_MODEL, H * E * dh).astype(jnp.bfloat16)
    wo_flat = Wo.reshape(H * E * dh, D_MODEL).astype(jnp.bfloat16)
    wsel = jnp.concatenate(
        [Wsel_v.reshape(D_MODEL, H, E).transpose(0, 2, 1).reshape(D_MODEL, E * H),
         Wsel_o.reshape(D_MODEL, H, E).transpose(0, 2, 1).reshape(D_MODEL, E * H)],
        axis=1)

    T = min(256, S)
    nt = S // T
    f32 = jnp.float32

    q, k, v, go = pl.pallas_call(
        functools.partial(_proj_kernel, tile=T),
        grid=(nt,),
        in_specs=[
            pl.BlockSpec((T, D_MODEL), lambda t: (t, 0)),
            pl.BlockSpec((D_MODEL, 2 * H * dh), lambda t: (0, 0)),
            pl.BlockSpec((D_MODEL, H * E * dh), lambda t: (0, 0)),
            pl.BlockSpec((D_MODEL, 2 * E * H), lambda t: (0, 0)),
        ],
        out_specs=[
            pl.BlockSpec((H, T, dh), lambda t: (0, t, 0)),
            pl.BlockSpec((H, T, dh), lambda t: (0, t, 0)),
            pl.BlockSpec((H, T, dh), lambda t: (0, t, 0)),
            pl.BlockSpec((T, E * H), lambda t: (t, 0)),
        ],
        out_shape=[
            jax.ShapeDtypeStruct((H, S, dh), f32),
            jax.ShapeDtypeStruct((H, S, dh), f32),
            jax.ShapeDtypeStruct((H, S, dh), f32),
            jax.ShapeDtypeStruct((S, E * H), f32),
        ],
    )(x, wqk, wv_flat, wsel)

    Tq = min(512, S)
    nq = S // Tq
    ctx = pl.pallas_call(
        functools.partial(_attn_kernel, tq=Tq, seq_len=S),
        grid=(H, nq),
        in_specs=[
            pl.BlockSpec((1, Tq, dh), lambda h, qt: (h, qt, 0)),
            pl.BlockSpec((1, S, dh), lambda h, qt: (h, 0, 0)),
            pl.BlockSpec((1, S, dh), lambda h, qt: (h, 0, 0)),
        ],
        out_specs=pl.BlockSpec((1, Tq, dh), lambda h, qt: (h, qt, 0)),
        out_shape=jax.ShapeDtypeStruct((H, S, dh), f32),
    )(q, k, v)

    out = pl.pallas_call(
        _out_kernel,
        grid=(nt,),
        in_specs=[
            pl.BlockSpec((H, T, dh), lambda t: (0, t, 0)),
            pl.BlockSpec((T, E * H), lambda t: (t, 0)),
            pl.BlockSpec((H * E * dh, D_MODEL), lambda t: (0, 0)),
        ],
        out_specs=pl.BlockSpec((T, D_MODEL), lambda t: (t, 0)),
        out_shape=jax.ShapeDtypeStruct((S, D_MODEL), f32),
        scratch_shapes=[pltpu.VMEM((T, H * E * dh), jnp.bfloat16)],
    )(ctx, go, wo_flat)

    return out.reshape(B, S, D_MODEL)


# speedup vs baseline: 1.1418x; 1.1418x over previous
import functools

import jax
import jax.numpy as jnp
import numpy as np
from jax.experimental import pallas as pl
from jax.experimental.pallas import tpu as pltpu

D_MODEL = 1024
N_HEADS = 16
D_HEAD = 64
N_EXP = 8
ROPE_BASE = 10000.0
NEG_INF = -1e30
HD = N_HEADS * D_HEAD
EHD = N_EXP * HD
EH = N_EXP * N_HEADS


def _top2_gates(logits_planes):
    m1 = logits_planes[0]
    i1 = jnp.zeros_like(m1)
    for e in range(1, N_EXP):
        gt = logits_planes[e] > m1
        m1 = jnp.where(gt, logits_planes[e], m1)
        i1 = jnp.where(gt, float(e), i1)
    p0 = jnp.where(i1 == 0.0, NEG_INF, logits_planes[0])
    m2 = p0
    i2 = jnp.zeros_like(m1)
    for e in range(1, N_EXP):
        pe = jnp.where(i1 == float(e), NEG_INF, logits_planes[e])
        gt = pe > m2
        m2 = jnp.where(gt, pe, m2)
        i2 = jnp.where(gt, float(e), i2)
    g1 = jax.nn.sigmoid(m1)
    g2 = jax.nn.sigmoid(m2)
    return [jnp.where(i1 == float(e), g1, 0.0) + jnp.where(i2 == float(e), g2, 0.0)
            for e in range(N_EXP)]


def _dot(a, b, trans_b=False):
    dims = (((1,), (1 if trans_b else 0,)), ((), ()))
    return jax.lax.dot_general(a, b, dims, preferred_element_type=jnp.float32)


def _proj_kernel(x_ref, wq4_ref, wv_ref, wsel_ref, rep_ref, cos_ref, sin_ref,
                 q_ref, k_ref, v_ref, go_ref):
    x = x_ref[...]
    xb = x.astype(jnp.bfloat16)

    qk4 = _dot(xb, wq4_ref[...])
    C = cos_ref[...]
    Sg = sin_ref[...]
    q_ref[...] = (qk4[:, :HD] * C + qk4[:, HD:2 * HD] * Sg).astype(jnp.bfloat16)
    k_ref[...] = (qk4[:, 2 * HD:3 * HD] * C + qk4[:, 3 * HD:] * Sg).astype(jnp.bfloat16)

    lg = _dot(x, wsel_ref[...])
    planes_v = [lg[:, e * N_HEADS:(e + 1) * N_HEADS] for e in range(N_EXP)]
    planes_o = [lg[:, EH + e * N_HEADS:EH + (e + 1) * N_HEADS] for e in range(N_EXP)]
    gates_v = _top2_gates(planes_v)
    gates_o = _top2_gates(planes_o)
    go_ref[...] = jnp.concatenate(gates_o, axis=1)
    gvb = jnp.concatenate(gates_v, axis=1).astype(jnp.bfloat16)

    vacc = None
    for e in range(N_EXP):
        ge = _dot(gvb, rep_ref[:, e * HD:(e + 1) * HD])
        ve = _dot(xb, wv_ref[:, e * HD:(e + 1) * HD])
        term = ve * ge
        vacc = term if vacc is None else vacc + term
    v_ref[...] = vacc.astype(jnp.bfloat16)


def _attn_kernel(q_ref, k_ref, v_ref, o_ref, *, tq, tk):
    qt = pl.program_id(0)
    scale = 1.0 / np.sqrt(D_HEAD)
    rows = jax.lax.broadcasted_iota(jnp.int32, (tq, tk), 0) + qt * tq
    cols0 = jax.lax.broadcasted_iota(jnp.int32, (tq, tk), 1)
    ntrips = (qt * tq) // tk + 1

    ctx_list = [None] * N_HEADS
    for h0 in range(0, N_HEADS, 2):
        qa = q_ref[h0]
        qb = q_ref[h0 + 1]

        def body(j, carry, h0=h0, qa=qa, qb=qb):
            ma, la, acca, mb, lb, accb = carry
            msk = cols0 + j * tk <= rows
            out = []
            for (q_, k_ref_h, m, l, acc) in ((qa, h0, ma, la, acca),
                                             (qb, h0 + 1, mb, lb, accb)):
                kj = k_ref[k_ref_h, pl.ds(j * tk, tk), :]
                vj = v_ref[k_ref_h, pl.ds(j * tk, tk), :]
                s = _dot(q_, kj, trans_b=True) * scale
                s = jnp.where(msk, s, NEG_INF)
                mn = jnp.maximum(m, jnp.max(s, axis=-1, keepdims=True))
                alpha = jnp.exp(m - mn)
                p = jnp.exp(s - mn)
                l2 = l * alpha + jnp.sum(p, axis=-1, keepdims=True)
                acc2 = acc * alpha + _dot(p.astype(jnp.bfloat16), vj)
                out += [mn, l2, acc2]
            return tuple(out)

        m0 = jnp.full((tq, 1), NEG_INF, jnp.float32)
        l0 = jnp.zeros((tq, 1), jnp.float32)
        a0 = jnp.zeros((tq, D_HEAD), jnp.float32)
        ma, la, acca, mb, lb, accb = jax.lax.fori_loop(
            0, ntrips, body, (m0, l0, a0, m0, l0, a0))
        ctx_list[h0] = acca / la
        ctx_list[h0 + 1] = accb / lb
    o_ref[...] = jnp.concatenate(ctx_list, axis=1)


def _out_kernel(ctx_ref, go_ref, rep_ref, wo_ref, o_ref):
    ctx = ctx_ref[...]
    gob = go_ref[...].astype(jnp.bfloat16)
    acc = None
    for e in range(N_EXP):
        ge = _dot(gob, rep_ref[:, e * HD:(e + 1) * HD])
        blk = (ctx * ge).astype(jnp.bfloat16)
        pe = _dot(blk, wo_ref[e * HD:(e + 1) * HD, :])
        acc = pe if acc is None else acc + pe
    o_ref[...] = acc


def kernel(token_stream, Wq, Wk, Wv, Wo, Wsel_v, Wsel_o):
    B, S, _ = token_stream.shape
    x = token_stream.reshape(S, D_MODEL)
    H, E, dh = N_HEADS, N_EXP, D_HEAD
    half = dh // 2
    f32 = jnp.float32
    bf16 = jnp.bfloat16

    def swap_halves(w):
        return w.reshape(D_MODEL, H, 2, half)[:, :, ::-1, :].reshape(D_MODEL, HD)

    wq4 = jnp.concatenate([Wq, swap_halves(Wq), Wk, swap_halves(Wk)], axis=1).astype(bf16)
    wv_em = Wv.transpose(2, 1, 0, 3).reshape(D_MODEL, EHD).astype(bf16)
    wo_em = Wo.transpose(1, 0, 2, 3).reshape(EHD, D_MODEL).astype(bf16)
    wsel = jnp.concatenate(
        [Wsel_v.reshape(D_MODEL, H, E).transpose(0, 2, 1).reshape(D_MODEL, EH),
         Wsel_o.reshape(D_MODEL, H, E).transpose(0, 2, 1).reshape(D_MODEL, EH)],
        axis=1)

    rcols = jnp.arange(EHD)
    e_c = rcols // HD
    h_c = (rcols % HD) // dh
    rep = (jnp.arange(EH)[:, None] == (e_c * H + h_c)[None, :]).astype(bf16)

    pos = jnp.arange(S, dtype=f32)
    inv_freq = 1.0 / (ROPE_BASE ** (jnp.arange(0, dh, 2, dtype=f32) / dh))
    ang = pos[:, None] * inv_freq[None, :]
    chead = jnp.concatenate([jnp.cos(ang), jnp.cos(ang)], axis=1)
    shead = jnp.concatenate([-jnp.sin(ang), jnp.sin(ang)], axis=1)
    ctab = jnp.tile(chead, (1, H))
    stab = jnp.tile(shead, (1, H))

    T = min(256, S)
    nt = S // T

    q, k, v, go = pl.pallas_call(
        _proj_kernel,
        grid=(nt,),
        in_specs=[
            pl.BlockSpec((T, D_MODEL), lambda t: (t, 0)),
            pl.BlockSpec((D_MODEL, 4 * HD), lambda t: (0, 0)),
            pl.BlockSpec((D_MODEL, EHD), lambda t: (0, 0)),
            pl.BlockSpec((D_MODEL, 2 * EH), lambda t: (0, 0)),
            pl.BlockSpec((EH, EHD), lambda t: (0, 0)),
            pl.BlockSpec((T, HD), lambda t: (t, 0)),
            pl.BlockSpec((T, HD), lambda t: (t, 0)),
        ],
        out_specs=[
            pl.BlockSpec((T, HD), lambda t: (t, 0)),
            pl.BlockSpec((T, HD), lambda t: (t, 0)),
            pl.BlockSpec((T, HD), lambda t: (t, 0)),
            pl.BlockSpec((T, EH), lambda t: (t, 0)),
        ],
        out_shape=[
            jax.ShapeDtypeStruct((S, HD), bf16),
            jax.ShapeDtypeStruct((S, HD), bf16),
            jax.ShapeDtypeStruct((S, HD), bf16),
            jax.ShapeDtypeStruct((S, EH), f32),
        ],
    )(x, wq4, wv_em, wsel, rep, ctab, stab)

    q3 = q.reshape(S, H, dh).transpose(1, 0, 2)
    k3 = k.reshape(S, H, dh).transpose(1, 0, 2)
    v3 = v.reshape(S, H, dh).transpose(1, 0, 2)

    Tq = min(512, S)
    Tk = min(512, S)
    nq = S // Tq
    ctx = pl.pallas_call(
        functools.partial(_attn_kernel, tq=Tq, tk=Tk),
        grid=(nq,),
        in_specs=[
            pl.BlockSpec((H, Tq, dh), lambda qt: (0, qt, 0)),
            pl.BlockSpec((H, S, dh), lambda qt: (0, 0, 0)),
            pl.BlockSpec((H, S, dh), lambda qt: (0, 0, 0)),
        ],
        out_specs=pl.BlockSpec((Tq, HD), lambda qt: (qt, 0)),
        out_shape=jax.ShapeDtypeStruct((S, HD), f32),
    )(q3, k3, v3)

    out = pl.pallas_call(
        _out_kernel,
        grid=(nt,),
        in_specs=[
            pl.BlockSpec((T, HD), lambda t: (t, 0)),
            pl.BlockSpec((T, EH), lambda t: (t, 0)),
            pl.BlockSpec((EH, EHD), lambda t: (0, 0)),
            pl.BlockSpec((EHD, D_MODEL), lambda t: (0, 0)),
        ],
        out_specs=pl.BlockSpec((T, D_MODEL), lambda t: (t, 0)),
        out_shape=jax.ShapeDtypeStruct((S, D_MODEL), f32),
    )(ctx, go, rep, wo_em)

    return out.reshape(B, S, D_MODEL)


# speedup vs baseline: 1.2496x; 1.0945x over previous
import functools

import jax
import jax.numpy as jnp
import numpy as np
from jax.experimental import pallas as pl
from jax.experimental.pallas import tpu as pltpu

D_MODEL = 1024
N_HEADS = 16
D_HEAD = 64
N_EXP = 8
ROPE_BASE = 10000.0
NEG_INF = -1e30
HD = N_HEADS * D_HEAD
EHD = N_EXP * HD
EH = N_EXP * N_HEADS


def _top2_gates(logits_planes):
    m1 = logits_planes[0]
    i1 = jnp.zeros_like(m1)
    for e in range(1, N_EXP):
        gt = logits_planes[e] > m1
        m1 = jnp.where(gt, logits_planes[e], m1)
        i1 = jnp.where(gt, float(e), i1)
    p0 = jnp.where(i1 == 0.0, NEG_INF, logits_planes[0])
    m2 = p0
    i2 = jnp.zeros_like(m1)
    for e in range(1, N_EXP):
        pe = jnp.where(i1 == float(e), NEG_INF, logits_planes[e])
        gt = pe > m2
        m2 = jnp.where(gt, pe, m2)
        i2 = jnp.where(gt, float(e), i2)
    g1 = jax.nn.sigmoid(m1)
    g2 = jax.nn.sigmoid(m2)
    return [jnp.where(i1 == float(e), g1, 0.0) + jnp.where(i2 == float(e), g2, 0.0)
            for e in range(N_EXP)]


def _dot(a, b, trans_b=False):
    dims = (((1,), (1 if trans_b else 0,)), ((), ()))
    return jax.lax.dot_general(a, b, dims, preferred_element_type=jnp.float32)


def _proj_kernel(x_ref, wq4_ref, wv_ref, wsel_ref, rep_ref, cos_ref, sin_ref,
                 q_ref, k_ref, v_ref, go_ref):
    x = x_ref[...]
    xb = x.astype(jnp.bfloat16)

    qk4 = _dot(xb, wq4_ref[...])
    C = cos_ref[...]
    Sg = sin_ref[...]
    q_ref[...] = (qk4[:, :HD] * C + qk4[:, HD:2 * HD] * Sg).astype(jnp.bfloat16)
    k_ref[...] = (qk4[:, 2 * HD:3 * HD] * C + qk4[:, 3 * HD:] * Sg).astype(jnp.bfloat16)

    lg = _dot(x, wsel_ref[...])
    planes_v = [lg[:, e * N_HEADS:(e + 1) * N_HEADS] for e in range(N_EXP)]
    planes_o = [lg[:, EH + e * N_HEADS:EH + (e + 1) * N_HEADS] for e in range(N_EXP)]
    gates_v = _top2_gates(planes_v)
    gates_o = _top2_gates(planes_o)
    go_ref[...] = jnp.concatenate(gates_o, axis=1)
    gvb = jnp.concatenate(gates_v, axis=1).astype(jnp.bfloat16)

    vacc = None
    for e in range(N_EXP):
        ge = _dot(gvb, rep_ref[:, e * HD:(e + 1) * HD])
        ve = _dot(xb, wv_ref[:, e * HD:(e + 1) * HD])
        term = ve * ge
        vacc = term if vacc is None else vacc + term
    v_ref[...] = vacc.astype(jnp.bfloat16)


def _attn_kernel(q_ref, k_ref, v_ref, o_ref, *, tq, tk):
    qt = pl.program_id(0)
    dh = D_HEAD
    rows = jax.lax.broadcasted_iota(jnp.int32, (tq, tk), 0) + qt * tq
    cols0 = jax.lax.broadcasted_iota(jnp.int32, (tq, tk), 1)
    ones = jnp.ones((tk, dh), jnp.bfloat16)
    m0 = jnp.full((tq, 1), NEG_INF, jnp.float32)
    a0 = jnp.zeros((tq, 2 * dh), jnp.float32)

    def step(q_, h, j, m, accl, masked):
        kj = k_ref[pl.ds(j * tk, tk), h * dh:(h + 1) * dh]
        vj = v_ref[pl.ds(j * tk, tk), h * dh:(h + 1) * dh]
        s = _dot(q_, kj, trans_b=True)
        if masked:
            s = jnp.where(cols0 + j * tk <= rows, s, NEG_INF)
        mn = jnp.maximum(m, jnp.max(s, axis=-1, keepdims=True))
        alpha = jnp.exp(m - mn)
        p = jnp.exp(s - mn)
        vjx = jnp.concatenate([vj, ones], axis=1)
        accl2 = accl * alpha + _dot(p.astype(jnp.bfloat16), vjx)
        return mn, accl2

    ctx_list = [None] * N_HEADS
    for h0 in range(0, N_HEADS, 2):
        qa = q_ref[:, h0 * dh:(h0 + 1) * dh]
        qb = q_ref[:, (h0 + 1) * dh:(h0 + 2) * dh]

        def body(j, carry, h0=h0, qa=qa, qb=qb):
            ma, accla, mb, acclb = carry
            ma, accla = step(qa, h0, j, ma, accla, masked=False)
            mb, acclb = step(qb, h0 + 1, j, mb, acclb, masked=False)
            return ma, accla, mb, acclb

        ma, accla, mb, acclb = jax.lax.fori_loop(0, qt, body, (m0, a0, m0, a0))
        ma, accla = step(qa, h0, qt, ma, accla, masked=True)
        mb, acclb = step(qb, h0 + 1, qt, mb, acclb, masked=True)
        ctx_list[h0] = accla[:, :dh] / accla[:, dh:dh + 1]
        ctx_list[h0 + 1] = acclb[:, :dh] / acclb[:, dh:dh + 1]
    o_ref[...] = jnp.concatenate(ctx_list, axis=1)


def _out_kernel(ctx_ref, go_ref, rep_ref, wo_ref, o_ref):
    ctx = ctx_ref[...]
    gob = go_ref[...].astype(jnp.bfloat16)
    acc = None
    for e in range(N_EXP):
        ge = _dot(gob, rep_ref[:, e * HD:(e + 1) * HD])
        blk = (ctx * ge).astype(jnp.bfloat16)
        pe = _dot(blk, wo_ref[e * HD:(e + 1) * HD, :])
        acc = pe if acc is None else acc + pe
    o_ref[...] = acc


def kernel(token_stream, Wq, Wk, Wv, Wo, Wsel_v, Wsel_o):
    B, S, _ = token_stream.shape
    x = token_stream.reshape(S, D_MODEL)
    H, E, dh = N_HEADS, N_EXP, D_HEAD
    half = dh // 2
    f32 = jnp.float32
    bf16 = jnp.bfloat16

    def swap_halves(w):
        return w.reshape(D_MODEL, H, 2, half)[:, :, ::-1, :].reshape(D_MODEL, HD)

    scale = 1.0 / np.sqrt(dh)
    wq4 = jnp.concatenate([Wq * scale, swap_halves(Wq) * scale,
                           Wk, swap_halves(Wk)], axis=1).astype(bf16)
    wv_em = Wv.transpose(2, 1, 0, 3).reshape(D_MODEL, EHD).astype(bf16)
    wo_em = Wo.transpose(1, 0, 2, 3).reshape(EHD, D_MODEL).astype(bf16)
    wsel = jnp.concatenate(
        [Wsel_v.reshape(D_MODEL, H, E).transpose(0, 2, 1).reshape(D_MODEL, EH),
         Wsel_o.reshape(D_MODEL, H, E).transpose(0, 2, 1).reshape(D_MODEL, EH)],
        axis=1)

    rcols = jnp.arange(EHD)
    e_c = rcols // HD
    h_c = (rcols % HD) // dh
    rep = (jnp.arange(EH)[:, None] == (e_c * H + h_c)[None, :]).astype(bf16)

    pos = jnp.arange(S, dtype=f32)
    inv_freq = 1.0 / (ROPE_BASE ** (jnp.arange(0, dh, 2, dtype=f32) / dh))
    ang = pos[:, None] * inv_freq[None, :]
    chead = jnp.concatenate([jnp.cos(ang), jnp.cos(ang)], axis=1)
    shead = jnp.concatenate([-jnp.sin(ang), jnp.sin(ang)], axis=1)
    ctab = jnp.tile(chead, (1, H))
    stab = jnp.tile(shead, (1, H))

    T = min(256, S)
    nt = S // T

    q, k, v, go = pl.pallas_call(
        _proj_kernel,
        grid=(nt,),
        in_specs=[
            pl.BlockSpec((T, D_MODEL), lambda t: (t, 0)),
            pl.BlockSpec((D_MODEL, 4 * HD), lambda t: (0, 0)),
            pl.BlockSpec((D_MODEL, EHD), lambda t: (0, 0)),
            pl.BlockSpec((D_MODEL, 2 * EH), lambda t: (0, 0)),
            pl.BlockSpec((EH, EHD), lambda t: (0, 0)),
            pl.BlockSpec((T, HD), lambda t: (t, 0)),
            pl.BlockSpec((T, HD), lambda t: (t, 0)),
        ],
        out_specs=[
            pl.BlockSpec((T, HD), lambda t: (t, 0)),
            pl.BlockSpec((T, HD), lambda t: (t, 0)),
            pl.BlockSpec((T, HD), lambda t: (t, 0)),
            pl.BlockSpec((T, EH), lambda t: (t, 0)),
        ],
        out_shape=[
            jax.ShapeDtypeStruct((S, HD), bf16),
            jax.ShapeDtypeStruct((S, HD), bf16),
            jax.ShapeDtypeStruct((S, HD), bf16),
            jax.ShapeDtypeStruct((S, EH), f32),
        ],
    )(x, wq4, wv_em, wsel, rep, ctab, stab)

    Tq = min(512, S)
    Tk = min(512, S)
    nq = S // Tq
    ctx = pl.pallas_call(
        functools.partial(_attn_kernel, tq=Tq, tk=Tk),
        grid=(nq,),
        in_specs=[
            pl.BlockSpec((Tq, HD), lambda qt: (qt, 0)),
            pl.BlockSpec((S, HD), lambda qt: (0, 0)),
            pl.BlockSpec((S, HD), lambda qt: (0, 0)),
        ],
        out_specs=pl.BlockSpec((Tq, HD), lambda qt: (qt, 0)),
        out_shape=jax.ShapeDtypeStruct((S, HD), f32),
    )(q, k, v)

    out = pl.pallas_call(
        _out_kernel,
        grid=(nt,),
        in_specs=[
            pl.BlockSpec((T, HD), lambda t: (t, 0)),
            pl.BlockSpec((T, EH), lambda t: (t, 0)),
            pl.BlockSpec((EH, EHD), lambda t: (0, 0)),
            pl.BlockSpec((EHD, D_MODEL), lambda t: (0, 0)),
        ],
        out_specs=pl.BlockSpec((T, D_MODEL), lambda t: (t, 0)),
        out_shape=jax.ShapeDtypeStruct((S, D_MODEL), f32),
    )(ctx, go, rep, wo_em)

    return out.reshape(B, S, D_MODEL)
